# Initial kernel scaffold; baseline (speedup 1.0000x reference)
import jax
import jax.numpy as jnp
from jax.experimental import pallas as pl


def kernel(x, edge_index, edge_attr, W0, b0, We0, be0, g0, beta0, W1, b1, We1, be1, g1, beta1):
    raise NotImplementedError("write your pallas kernel here")


# speedup vs baseline: 8.7241x; 8.7241x over previous
import functools

import jax
import jax.numpy as jnp
from jax import lax
from jax.experimental import pallas as pl
from jax.experimental.pallas import tpu as pltpu
from jax.experimental.pallas import tpu_sc as plsc

NC = 2
NS = 16
NW = NC * NS
C = 80
NB = 5

_mesh = functools.partial(
    plsc.VectorSubcoreMesh, core_axis_name="c", subcore_axis_name="s")


def _pad_rows(n):
    blk = NS * C * 8
    return ((n + blk - 1) // blk) * blk


def _zero_fill(ref, nrows, ncols):
    zero16 = jnp.zeros((16,), jnp.float32)

    def body(i, carry):
        for c in range(ncols // 16):
            ref[i, pl.ds(c * 16, 16)] = zero16
        return carry

    lax.fori_loop(0, nrows, body, 0)


@functools.lru_cache(maxsize=None)
def _deg_kernel(N, E):
    NP = _pad_rows(N)
    NCH = E // (NW * C)
    DB = NCH // NB
    RPT = NP // NS

    @functools.partial(
        pl.kernel,
        out_type=jax.ShapeDtypeStruct((NC, NP, 128), jnp.float32),
        mesh=_mesh(),
        scratch_types=[
            pltpu.VMEM((DB, C), jnp.int32),
            pltpu.VMEM((C, 128), jnp.float32),
            pltpu.VMEM((C, 128), jnp.float32),
            pltpu.VMEM_SHARED((NP, 128), jnp.float32),
        ],
    )
    def k(dst_hbm, out_hbm, dst_v, ones_v, zb_v, acc_sh):
        cid = lax.axis_index("c")
        sid = lax.axis_index("s")
        wid = cid * NS + sid
        _zero_fill(zb_v, C, 128)
        one16 = jnp.ones((16,), jnp.float32)

        def fill_ones(i, carry):
            for cc in range(8):
                ones_v[i, pl.ds(cc * 16, 16)] = one16
            return carry

        lax.fori_loop(0, C, fill_ones, 0)
        row0 = sid * RPT
        for kk in range(RPT // C):
            pltpu.sync_copy(zb_v, acc_sh.at[pl.ds(row0 + kk * C, C)])
        plsc.subcore_barrier()

        def inner(jj, b):
            pltpu.sync_copy(ones_v, acc_sh.at[dst_v.at[jj]], add=True)
            return b

        def outer(b, carry):
            pltpu.sync_copy(dst_hbm.at[wid, b], dst_v)
            lax.fori_loop(0, DB, inner, b)
            return carry

        lax.fori_loop(0, NB, outer, 0)
        plsc.subcore_barrier()
        pltpu.sync_copy(acc_sh.at[pl.ds(row0, RPT)],
                        out_hbm.at[cid, pl.ds(row0, RPT)])

    return k


@functools.lru_cache(maxsize=None)
def _attr_kernel(N, E, DE):
    NP = _pad_rows(N)
    NCH = E // (NW * C)
    DB = NCH // NB
    EPW = E // NW
    RPT = NP // NS
    DE2 = 2 * DE

    @functools.partial(
        pl.kernel,
        out_type=jax.ShapeDtypeStruct((NC, NP, 128), jnp.float32),
        mesh=_mesh(),
        scratch_types=[
            pltpu.VMEM((DB, C), jnp.int32),
            pltpu.VMEM((DB, C), jnp.int32),
            pltpu.VMEM((C, DE), jnp.float32),
            pltpu.VMEM((C, 128), jnp.float32),
            pltpu.VMEM((N + 16,), jnp.float32),
            pltpu.VMEM_SHARED((NP, 128), jnp.float32),
        ],
    )
    def k(src_hbm, dst_hbm, ea_hbm, dinv_hbm, t_out,
          src_v, dst_v, ea_v, dv_v, dinv_v, tacc):
        cid = lax.axis_index("c")
        sid = lax.axis_index("s")
        wid = cid * NS + sid
        pltpu.sync_copy(dinv_hbm, dinv_v.at[pl.ds(0, N)])
        _zero_fill(dv_v, C, 128)
        row0 = sid * RPT
        for kk in range(RPT // C):
            pltpu.sync_copy(dv_v, tacc.at[pl.ds(row0 + kk * C, C)])
        plsc.subcore_barrier()

        def grp_body(g, jj):
            src16 = src_v[jj, pl.ds(g * 16, 16)]
            for t in range(16):
                v = dinv_v[pl.ds(src16[t], 16)]
                d16 = jnp.full((16,), v[0], jnp.float32)
                e = g * 16 + t
                dv_v[e, pl.ds(0, DE)] = ea_v[e, :] * d16
                dv_v[e, pl.ds(DE, DE)] = d16
            return jj

        def inner(jj, b):
            j = b * DB + jj
            base_e = (wid * NCH + j) * C
            pltpu.sync_copy(ea_hbm.at[pl.ds(base_e, C)], ea_v)
            lax.fori_loop(0, C // 16, grp_body, jj)
            pltpu.sync_copy(dv_v, tacc.at[dst_v.at[jj]], add=True)
            return b

        def outer(b, carry):
            pltpu.sync_copy(src_hbm.at[wid, b], src_v)
            pltpu.sync_copy(dst_hbm.at[wid, b], dst_v)
            lax.fori_loop(0, DB, inner, b)
            return carry

        lax.fori_loop(0, NB, outer, 0)
        plsc.subcore_barrier()
        pltpu.sync_copy(tacc.at[pl.ds(row0, RPT)],
                        t_out.at[cid, pl.ds(row0, RPT)])

    return k


@functools.lru_cache(maxsize=None)
def _gather_kernel(N, E, DN):
    NP = _pad_rows(N)
    NCH = E // (NW * C)
    DB = NCH // NB
    EPW = E // NW
    RPT = NP // NS

    @functools.partial(
        pl.kernel,
        out_type=jax.ShapeDtypeStruct((NC, NP, DN), jnp.float32),
        mesh=_mesh(),
        scratch_types=[
            pltpu.VMEM((DB, C), jnp.int32),
            pltpu.VMEM((DB, C), jnp.int32),
            pltpu.VMEM((C, DN), jnp.float32),
            pltpu.VMEM_SHARED((NP, DN), jnp.float32),
            pltpu.SemaphoreType.DMA,
        ],
    )
    def k(src_hbm, dst_hbm, hd_hbm, s_out,
          src_v, dst_v, rows_v, sacc, sem):
        cid = lax.axis_index("c")
        sid = lax.axis_index("s")
        wid = cid * NS + sid
        _zero_fill(rows_v, C, DN)
        row0 = sid * RPT
        for kk in range(RPT // C):
            pltpu.sync_copy(rows_v, sacc.at[pl.ds(row0 + kk * C, C)])
        plsc.subcore_barrier()

        def inner(jj, b):
            pltpu.async_copy(
                hd_hbm.at[src_v.at[jj]], rows_v, sem).wait()
            pltpu.sync_copy(rows_v, sacc.at[dst_v.at[jj]], add=True)
            return b

        def outer(b, carry):
            pltpu.sync_copy(src_hbm.at[wid, b], src_v)
            pltpu.sync_copy(dst_hbm.at[wid, b], dst_v)
            lax.fori_loop(0, DB, inner, b)
            return carry

        lax.fori_loop(0, NB, outer, 0)
        plsc.subcore_barrier()
        pltpu.sync_copy(sacc.at[pl.ds(row0, RPT)],
                        s_out.at[cid, pl.ds(row0, RPT)])

    return k


def _prep_body(N, d0, d1, x, dinv1, dinv, hd):
    deg = jnp.maximum(d0[...] + d1[...], 1.0)[:, :1]
    dv = lax.rsqrt(deg)
    dinv1[...] = jnp.squeeze(dv[:N], axis=1)
    dinv[...] = dv
    hd[...] = dv[:N] * x[...]


def _mm_body(DE, S_in, T_in, dinv, W, We, b, be, z_ref):
    S = S_in[...]
    Tm = T_in[...]
    hp = lax.Precision.HIGHEST
    z_ref[...] = dinv[...] * (
        jnp.dot(S, W[...], preferred_element_type=jnp.float32, precision=hp)
        + jnp.dot(Tm[:, :DE], We[...],
                  preferred_element_type=jnp.float32, precision=hp)
        + Tm[:, DE:DE + 1] * (b[...] + be[...]))


def _bn_body(N, relu, out_hd, z, dinv, g, beta, *outs):
    zz = z[...][:N]
    mu = jnp.mean(zz, axis=0, keepdims=True)
    cen = zz - mu
    var = jnp.mean(cen * cen, axis=0, keepdims=True)
    h = g[...] * cen * lax.rsqrt(var + 1e-5) + beta[...]
    if relu:
        h = jnp.maximum(h, 0.0)
    outs[0][...] = h
    if out_hd:
        outs[1][...] = dinv[...][:N] * h


def kernel(x, edge_index, edge_attr, W0, b0, We0, be0, g0, beta0,
           W1, b1, We1, be1, g1, beta1):
    N, DN = x.shape
    E, DE = edge_attr.shape
    H = W0.shape[1]
    f32 = jnp.float32
    NCH = E // (NW * C)
    DB = NCH // NB
    NPad = _pad_rows(N)

    src_r = edge_index[0].reshape(NW, NB, DB, C)
    dst_r = edge_index[1].reshape(NW, NB, DB, C)

    degp = _deg_kernel(N, E)(dst_r)

    dinv1, dinvP, hd0 = pl.pallas_call(
        functools.partial(_prep_body, N),
        out_shape=[jax.ShapeDtypeStruct((N,), f32),
                   jax.ShapeDtypeStruct((NPad, 1), f32),
                   jax.ShapeDtypeStruct((N, DN), f32)],
    )(degp[0], degp[1], x)

    tp = _attr_kernel(N, E, DE)(src_r, dst_r, edge_attr, dinv1)
    s0p = _gather_kernel(N, E, DN)(src_r, dst_r, hd0)

    r2 = lambda v: v.reshape(1, H)
    GB = 8
    BM = NPad // GB

    def dense(relu, out_hd, sp_, tp_, Wl, Wel, bl, bel, gl, betal):
        S_in = sp_[0] + sp_[1]
        T_in = tp_[0] + tp_[1]
        bm = lambda i: (i, 0)
        z0 = lambda i: (0, 0)
        Z = pl.pallas_call(
            functools.partial(_mm_body, DE),
            grid=(GB,),
            in_specs=[pl.BlockSpec((BM, DN), bm),
                      pl.BlockSpec((BM, 128), bm),
                      pl.BlockSpec((BM, 1), bm),
                      pl.BlockSpec((DN, H), z0),
                      pl.BlockSpec((DE, H), z0),
                      pl.BlockSpec((1, H), z0),
                      pl.BlockSpec((1, H), z0)],
            out_specs=pl.BlockSpec((BM, H), bm),
            out_shape=jax.ShapeDtypeStruct((NPad, H), f32),
        )(S_in, T_in, dinvP, Wl, Wel, r2(bl), r2(bel))
        shapes = [jax.ShapeDtypeStruct((N, H), f32)]
        if out_hd:
            shapes.append(jax.ShapeDtypeStruct((N, H), f32))
        return pl.pallas_call(
            functools.partial(_bn_body, N, relu, out_hd),
            out_shape=shapes,
        )(Z, dinvP, r2(gl), r2(betal))

    h1, hd1 = dense(True, True, s0p, tp, W0, We0, b0, be0, g0, beta0)
    s1p = _gather_kernel(N, E, H)(src_r, dst_r, hd1)
    (h2,) = dense(False, False, s1p, tp, W1, We1, b1, be1, g1, beta1)
    return h2
